# baseline (speedup 1.0000x reference)
import numpy as np

import jax
import jax.numpy as jnp
from jax import lax
from jax.experimental import pallas as pl
from jax.experimental.pallas import tpu as pltpu
from jax.experimental.pallas import tpu_sc as plsc

ROWS = 128
N = 32768
L = 16
NV = N // L
NC = 2
NS = 16
NW = NC * NS
RPW = ROWS // NW

NB = 2048
SHIFTS = (0, 11, 22)
MASKS = (2047, 2047, 1023)
NBINS = (2048, 2048, 1024)

UNROLL = 16

MININT = np.int32(-2147483648)


def _to_key(v):
    m = v >> 31
    return v ^ (m | MININT)


def _from_key(k):
    m = k >> 31
    return k ^ (~m | MININT)


def _digit(k, p):
    return lax.shift_right_logical(k, jnp.int32(SHIFTS[p])) & jnp.int32(MASKS[p])


def _zero(ref, n):
    zeros = jnp.zeros((L,), jnp.int32)

    def body(i, c):
        ref[pl.ds(i * L, L)] = zeros
        return c

    lax.fori_loop(0, n // L, body, 0)


def _body(x_hbm, out_hbm, buf_a, buf_b, rep, h0, h1, h2):
    wid = lax.axis_index("s") * NC + lax.axis_index("c")
    row0 = wid * RPW
    lanebase = lax.iota(jnp.int32, 16) * jnp.int32(NB)
    one = jnp.full((L,), 1, jnp.int32)

    _zero(rep, 16 * NB)

    def merge_excl(hist, nbins):
        zeros = jnp.zeros((L,), jnp.int32)

        def body(i, carry):
            ds = pl.ds(i * L, L)
            parts = [rep[pl.ds(j * NB + i * L, L)] for j in range(16)]
            tot = parts[0]
            for x in parts[1:]:
                tot = tot + x
            inc = plsc.cumsum(tot)
            hist[ds] = inc - tot + carry
            for j in range(16):
                rep[pl.ds(j * NB + i * L, L)] = zeros
            return carry + jnp.max(inc)

        lax.fori_loop(0, nbins // L, body, jnp.int32(0))

    def count0(src):
        def sweep(i, c):
            ks = [_to_key(src[pl.ds((i * UNROLL + u) * L, L)])
                  for u in range(UNROLL)]
            for k in ks:
                plsc.addupdate_scatter(rep, [lanebase + _digit(k, 0)], one)
            return c

        lax.fori_loop(0, NV // UNROLL, sweep, 0)

    def permute(p, src, dst, hist, count_p):
        def sweep(i, c):
            raw = [src[pl.ds((i * UNROLL + u) * L, L)]
                   for u in range(UNROLL)]
            ks = [_to_key(v) for v in raw] if p == 0 else raw
            digs = [_digit(k, p) for k in ks]
            scans = [plsc.scan_count(d) for d in digs]
            vals = ks if p < 2 else [_from_key(k) for k in ks]
            for u in range(UNROLL):
                cnt, lastm = scans[u]
                d = digs[u]
                base = plsc.load_gather(hist, [d])
                off = base + cnt - 1
                plsc.store_scatter(dst, [off], vals[u])
                plsc.store_scatter(hist, [d], base + cnt, mask=lastm)
            if count_p is not None:
                for k in ks:
                    plsc.addupdate_scatter(
                        rep, [lanebase + _digit(k, count_p)], one)
            return c

        lax.fori_loop(0, NV // UNROLL, sweep, 0)

    pltpu.sync_copy(x_hbm.at[row0], buf_a)
    count0(buf_a)
    for r in range(RPW):
        merge_excl(h0, NBINS[0])
        permute(0, buf_a, buf_b, h0, 1)
        merge_excl(h1, NBINS[1])
        permute(1, buf_b, buf_a, h1, 2)
        merge_excl(h2, NBINS[2])
        permute(2, buf_a, buf_b, h2, None)
        pltpu.sync_copy(buf_b, out_hbm.at[row0 + r])
        if r + 1 < RPW:
            pltpu.sync_copy(x_hbm.at[row0 + (r + 1)], buf_a)
            count0(buf_a)


@jax.jit
def kernel(x):
    xi = lax.bitcast_convert_type(x, jnp.int32)
    mesh = plsc.VectorSubcoreMesh(core_axis_name="c", subcore_axis_name="s")
    sort_rows = pl.kernel(
        _body,
        out_type=jax.ShapeDtypeStruct((ROWS, N), jnp.int32),
        mesh=mesh,
        compiler_params=pltpu.CompilerParams(needs_layout_passes=False),
        scratch_types=[
            pltpu.VMEM((N,), jnp.int32),
            pltpu.VMEM((N,), jnp.int32),
            pltpu.VMEM((16 * NB,), jnp.int32),
            pltpu.VMEM((NBINS[0],), jnp.int32),
            pltpu.VMEM((NBINS[1],), jnp.int32),
            pltpu.VMEM((NBINS[2],), jnp.int32),
        ],
    )
    oi = sort_rows(xi)
    return lax.bitcast_convert_type(oi, jnp.float32)


# speedup vs baseline: 1.1575x; 1.1575x over previous
import numpy as np

import jax
import jax.numpy as jnp
from jax import lax
from jax.experimental import pallas as pl
from jax.experimental.pallas import tpu as pltpu
from jax.experimental.pallas import tpu_sc as plsc

ROWS = 128
N = 32768
L = 16
NV = N // L
NC = 2
NS = 16
NW = NC * NS
RPW = ROWS // NW

NB = 2048
SHIFTS = (0, 11, 22)
MASKS = (2047, 2047, 1023)
NBINS = (2048, 2048, 1024)

MININT = np.int32(-2147483648)


def _to_key(v):
    m = v >> 31
    return v ^ (m | MININT)


def _from_key(k):
    m = k >> 31
    return k ^ (~m | MININT)


def _digit(k, p):
    return lax.shift_right_logical(k, jnp.int32(SHIFTS[p])) & jnp.int32(MASKS[p])


def _zero_hist(hist, nbins):
    zeros = jnp.zeros((L,), jnp.int32)

    def body(i, c):
        hist[pl.ds(i * L, L)] = zeros
        return c

    lax.fori_loop(0, nbins // L, body, 0)


def _exclusive_scan(hist, nbins):
    def body(i, carry):
        h = hist[pl.ds(i * L, L)]
        inc = plsc.cumsum(h)
        hist[pl.ds(i * L, L)] = inc - h + carry
        return carry + jnp.sum(h)

    lax.fori_loop(0, nbins // L, body, jnp.int32(0))


UNROLL = 16


def _body(x_hbm, out_hbm, buf_a, buf_b, buf_c, hist_0, hist_1, hist_2,
          sem_in, sem_out):
    wid = lax.axis_index("s") * NC + lax.axis_index("c")
    hists = (hist_0, hist_1, hist_2)
    bufs = (buf_a, buf_b, buf_c)
    row0 = wid * RPW

    def sort_row(src0, pong):
        for p in range(3):
            _zero_hist(hists[p], NBINS[p])

        def sweep0(i, c):
            ks = []
            for u in range(UNROLL):
                v = src0[pl.ds((i * UNROLL + u) * L, L)]
                ks.append(_to_key(v))
            digs = [[_digit(k, p) for k in ks] for p in range(3)]
            for p in range(3):
                scans = [plsc.scan_count(d) for d in digs[p]]
                for u in range(UNROLL):
                    cnt, lastm = scans[u]
                    plsc.addupdate_scatter(hists[p], [digs[p][u]], cnt,
                                           mask=lastm)
            return c

        lax.fori_loop(0, NV // UNROLL, sweep0, 0)

        def permute(p, src, dst):
            hist = hists[p]
            _exclusive_scan(hist, NBINS[p])

            def sweep(i, c):
                raw = [src[pl.ds((i * UNROLL + u) * L, L)]
                       for u in range(UNROLL)]
                ks = [_to_key(v) for v in raw] if p == 0 else raw
                digs = [_digit(k, p) for k in ks]
                scans = [plsc.scan_count(d) for d in digs]
                vals = ks if p < 2 else [_from_key(k) for k in ks]
                for u in range(UNROLL):
                    cnt, lastm = scans[u]
                    d = digs[u]
                    base = plsc.load_gather(hist, [d])
                    nxt = base + cnt
                    plsc.store_scatter(hist, [d], nxt, mask=lastm)
                    plsc.store_scatter(dst, [nxt - 1], vals[u])
                return c

            lax.fori_loop(0, NV // UNROLL, sweep, 0)

        permute(0, src0, pong)
        permute(1, pong, src0)
        permute(2, src0, pong)

    sched_x = [0, 2, 1, 0]
    sched_y = [1, 0, 2, 1]
    in_h = {0: pltpu.async_copy(x_hbm.at[row0], bufs[0], sem_in)}
    out_h = {}
    for r in range(RPW):
        x_buf = bufs[sched_x[r]]
        y_buf = bufs[sched_y[r]]
        in_h[r].wait()
        if r >= 1:
            out_h[r - 1].wait()
        if r + 1 < RPW:
            in_h[r + 1] = pltpu.async_copy(
                x_hbm.at[row0 + (r + 1)], bufs[sched_x[r + 1]], sem_in)
        sort_row(x_buf, y_buf)
        out_h[r] = pltpu.async_copy(y_buf, out_hbm.at[row0 + r], sem_out)
    out_h[RPW - 1].wait()


@jax.jit
def kernel(x):
    xi = lax.bitcast_convert_type(x, jnp.int32)
    mesh = plsc.VectorSubcoreMesh(core_axis_name="c", subcore_axis_name="s")
    sort_rows = pl.kernel(
        _body,
        out_type=jax.ShapeDtypeStruct((ROWS, N), jnp.int32),
        mesh=mesh,
        compiler_params=pltpu.CompilerParams(needs_layout_passes=False),
        scratch_types=[
            pltpu.VMEM((N,), jnp.int32),
            pltpu.VMEM((N,), jnp.int32),
            pltpu.VMEM((N,), jnp.int32),
            pltpu.VMEM((NBINS[0],), jnp.int32),
            pltpu.VMEM((NBINS[1],), jnp.int32),
            pltpu.VMEM((NBINS[2],), jnp.int32),
            pltpu.SemaphoreType.DMA,
            pltpu.SemaphoreType.DMA,
        ],
    )
    oi = sort_rows(xi)
    return lax.bitcast_convert_type(oi, jnp.float32)


# speedup vs baseline: 1.1826x; 1.0217x over previous
import numpy as np

import jax
import jax.numpy as jnp
from jax import lax
from jax.experimental import pallas as pl
from jax.experimental.pallas import tpu as pltpu
from jax.experimental.pallas import tpu_sc as plsc

ROWS = 128
N = 32768
L = 16
NV = N // L
NC = 2
NS = 16
NW = NC * NS
RPW = ROWS // NW

NB = 2048
SHIFTS = (0, 11, 22)
MASKS = (2047, 2047, 1023)
NBINS = (2048, 2048, 1024)

MININT = np.int32(-2147483648)


def _to_key(v):
    m = v >> 31
    return v ^ (m | MININT)


def _from_key(k):
    m = k >> 31
    return k ^ (~m | MININT)


def _digit(k, p):
    return lax.shift_right_logical(k, jnp.int32(SHIFTS[p])) & jnp.int32(MASKS[p])


def _zero_hist(hist, nbins):
    zeros = jnp.zeros((L,), jnp.int32)

    def body(i, c):
        hist[pl.ds(i * L, L)] = zeros
        return c

    lax.fori_loop(0, nbins // L, body, 0)


def _exclusive_scan(hist, nbins):
    def body(i, carry):
        h = hist[pl.ds(i * L, L)]
        inc = plsc.cumsum(h)
        hist[pl.ds(i * L, L)] = inc - h + carry
        return carry + jnp.sum(h)

    lax.fori_loop(0, nbins // L, body, jnp.int32(0))


UNROLL = 32


def _body(x_hbm, out_hbm, buf_a, buf_b, buf_c, hist_0, hist_1, hist_2,
          sem_in, sem_out):
    wid = lax.axis_index("s") * NC + lax.axis_index("c")
    hists = (hist_0, hist_1, hist_2)
    bufs = (buf_a, buf_b, buf_c)
    row0 = wid * RPW

    def sort_row(src0, pong):
        for p in range(3):
            _zero_hist(hists[p], NBINS[p])

        def sweep0(i, c):
            ks = []
            for u in range(UNROLL):
                v = src0[pl.ds((i * UNROLL + u) * L, L)]
                ks.append(_to_key(v))
            digs = [[_digit(k, p) for k in ks] for p in range(3)]
            for p in range(3):
                scans = [plsc.scan_count(d) for d in digs[p]]
                for u in range(UNROLL):
                    cnt, lastm = scans[u]
                    plsc.addupdate_scatter(hists[p], [digs[p][u]], cnt,
                                           mask=lastm)
            return c

        lax.fori_loop(0, NV // UNROLL, sweep0, 0)

        def permute(p, src, dst):
            hist = hists[p]
            _exclusive_scan(hist, NBINS[p])

            def sweep(i, c):
                raw = [src[pl.ds((i * UNROLL + u) * L, L)]
                       for u in range(UNROLL)]
                ks = [_to_key(v) for v in raw] if p == 0 else raw
                digs = [_digit(k, p) for k in ks]
                scans = [plsc.scan_count(d) for d in digs]
                vals = ks if p < 2 else [_from_key(k) for k in ks]
                for u in range(UNROLL):
                    cnt, lastm = scans[u]
                    d = digs[u]
                    base = plsc.load_gather(hist, [d])
                    nxt = base + cnt
                    plsc.store_scatter(hist, [d], nxt, mask=lastm)
                    plsc.store_scatter(dst, [nxt - 1], vals[u])
                return c

            lax.fori_loop(0, NV // UNROLL, sweep, 0)

        permute(0, src0, pong)
        permute(1, pong, src0)
        permute(2, src0, pong)

    sched_x = [0, 2, 1, 0]
    sched_y = [1, 0, 2, 1]
    in_h = {0: pltpu.async_copy(x_hbm.at[row0], bufs[0], sem_in)}
    out_h = {}
    for r in range(RPW):
        x_buf = bufs[sched_x[r]]
        y_buf = bufs[sched_y[r]]
        in_h[r].wait()
        if r >= 1:
            out_h[r - 1].wait()
        if r + 1 < RPW:
            in_h[r + 1] = pltpu.async_copy(
                x_hbm.at[row0 + (r + 1)], bufs[sched_x[r + 1]], sem_in)
        sort_row(x_buf, y_buf)
        out_h[r] = pltpu.async_copy(y_buf, out_hbm.at[row0 + r], sem_out)
    out_h[RPW - 1].wait()


@jax.jit
def kernel(x):
    xi = lax.bitcast_convert_type(x, jnp.int32)
    mesh = plsc.VectorSubcoreMesh(core_axis_name="c", subcore_axis_name="s")
    sort_rows = pl.kernel(
        _body,
        out_type=jax.ShapeDtypeStruct((ROWS, N), jnp.int32),
        mesh=mesh,
        compiler_params=pltpu.CompilerParams(needs_layout_passes=False),
        scratch_types=[
            pltpu.VMEM((N,), jnp.int32),
            pltpu.VMEM((N,), jnp.int32),
            pltpu.VMEM((N,), jnp.int32),
            pltpu.VMEM((NBINS[0],), jnp.int32),
            pltpu.VMEM((NBINS[1],), jnp.int32),
            pltpu.VMEM((NBINS[2],), jnp.int32),
            pltpu.SemaphoreType.DMA,
            pltpu.SemaphoreType.DMA,
        ],
    )
    oi = sort_rows(xi)
    return lax.bitcast_convert_type(oi, jnp.float32)
